# Initial kernel scaffold; baseline (speedup 1.0000x reference)
import jax
import jax.numpy as jnp
from jax.experimental import pallas as pl


def kernel(attr, metric_scores):
    raise NotImplementedError("write your pallas kernel here")


# speedup vs baseline: 1.1008x; 1.1008x over previous
import math

import jax
import jax.numpy as jnp
from jax import lax
from jax.experimental import pallas as pl
from jax.experimental.pallas import tpu as pltpu
from jax.experimental.pallas import tpu_sc as plsc

_BETA = 0.9
_LN_BETA = math.log(_BETA)

_B = 16384 * 26
_NC, _NS = 2, 16
_NW = _NC * _NS
_BPW = _B // _NW
_L = 16


def _fair_scaler_body(attr_hbm, ms_hbm, out_hbm, idx_v, vals_v, sem):
    wid = lax.axis_index("s") * _NC + lax.axis_index("c")
    base = wid * _BPW
    pltpu.sync_copy(attr_hbm.at[pl.ds(base, _BPW)], idx_v)
    pltpu.async_copy(ms_hbm.at[idx_v], vals_v, sem).wait()

    def step(i, carry):
        n = vals_v[pl.ds(i * _L, _L)]
        w = (1.0 - _BETA) / (1.0 - jnp.exp(n * _LN_BETA))
        vals_v[pl.ds(i * _L, _L)] = w
        return carry

    lax.fori_loop(0, _BPW // _L, step, 0)
    pltpu.sync_copy(vals_v, out_hbm.at[pl.ds(base, _BPW)])


_sc_call = pl.kernel(
    _fair_scaler_body,
    mesh=plsc.VectorSubcoreMesh(core_axis_name="c", subcore_axis_name="s"),
    out_type=jax.ShapeDtypeStruct((_B,), jnp.float32),
    scratch_types=[
        pltpu.VMEM((_BPW,), jnp.int32),
        pltpu.VMEM((_BPW,), jnp.float32),
        pltpu.SemaphoreType.DMA,
    ],
)


def kernel(attr, metric_scores):
    out = _sc_call(attr.reshape(-1), metric_scores)
    return out.reshape(attr.shape)


# speedup vs baseline: 1.2341x; 1.1211x over previous
import math

import jax
import jax.numpy as jnp
from jax import lax
from jax.experimental import pallas as pl
from jax.experimental.pallas import tpu as pltpu
from jax.experimental.pallas import tpu_sc as plsc

_BETA = 0.9
_LN_BETA = math.log(_BETA)

_ROWS, _COLS = 16384, 26
_NC, _NS = 2, 16
_NW = _NC * _NS
_RPW = _ROWS // _NW
_EPW = _RPW * _COLS
_L = 16
_RC = 64
_NCHUNK = _RPW // _RC
_EPC = _RC * _COLS
_SPC = _EPC // _L


def _advance(row, col):
    col = col + _L
    wrap = col >= _COLS
    col = jnp.where(wrap, col - _COLS, col)
    row = jnp.where(wrap, row + 1, row)
    return row, col


def _fair_scaler_body(attr_hbm, ms_hbm, out_hbm, idx2_v, idx_v, vals_v,
                      out2_v, sem):
    wid = lax.axis_index("s") * _NC + lax.axis_index("c")
    r0 = wid * _RPW
    col0 = lax.iota(jnp.int32, _L)
    row0 = jnp.zeros((_L,), jnp.int32)

    for c in range(_NCHUNK):
        pltpu.sync_copy(attr_hbm.at[pl.ds(r0 + c * _RC, _RC)], idx2_v)

        def flatten(k, carry, c=c):
            row, col = carry
            idx_v[pl.ds(c * _EPC + k * _L, _L)] = plsc.load_gather(
                idx2_v, [row, col])
            return _advance(row, col)

        lax.fori_loop(0, _SPC, flatten, (row0, col0))

    pltpu.async_copy(ms_hbm.at[idx_v], vals_v, sem).wait()

    for c in range(_NCHUNK):
        def transform(k, carry, c=c):
            row, col = carry
            n = vals_v[pl.ds(c * _EPC + k * _L, _L)]
            w = (1.0 - _BETA) / (1.0 - jnp.exp(n * _LN_BETA))
            plsc.store_scatter(out2_v, [row, col], w)
            return _advance(row, col)

        lax.fori_loop(0, _SPC, transform, (row0, col0))
        pltpu.sync_copy(out2_v, out_hbm.at[pl.ds(r0 + c * _RC, _RC)])


_sc_call = pl.kernel(
    _fair_scaler_body,
    mesh=plsc.VectorSubcoreMesh(core_axis_name="c", subcore_axis_name="s"),
    out_type=jax.ShapeDtypeStruct((_ROWS, _COLS), jnp.float32),
    compiler_params=pltpu.CompilerParams(needs_layout_passes=False),
    scratch_types=[
        pltpu.VMEM((_RC, _COLS), jnp.int32),
        pltpu.VMEM((_EPW,), jnp.int32),
        pltpu.VMEM((_EPW,), jnp.float32),
        pltpu.VMEM((_RC, _COLS), jnp.float32),
        pltpu.SemaphoreType.DMA,
    ],
)


def kernel(attr, metric_scores):
    return _sc_call(attr, metric_scores)


# speedup vs baseline: 1.7389x; 1.4090x over previous
import math

import jax
import jax.numpy as jnp
from jax import lax
from jax.experimental import pallas as pl
from jax.experimental.pallas import tpu as pltpu
from jax.experimental.pallas import tpu_sc as plsc

_BETA = 0.9
_LN_BETA = math.log(_BETA)

_N, _A = 16384, 26
_NC, _NS = 2, 16
_NW = _NC * _NS
_CPW = _N // _NW
_EPW = _CPW * _A
_L = 16


def _fair_scaler_body(attr_hbm, ms_hbm, out_hbm, idx_v, vals_v, sem):
    wid = lax.axis_index("s") * _NC + lax.axis_index("c")
    c0 = wid * _CPW
    copies = [
        pltpu.make_async_copy(
            attr_hbm.at[a, pl.ds(c0, _CPW)],
            idx_v.at[pl.ds(a * _CPW, _CPW)],
            sem,
        )
        for a in range(_A)
    ]
    for c in copies:
        c.start()
    for c in copies:
        c.wait()

    pltpu.async_copy(ms_hbm.at[idx_v], vals_v, sem).wait()

    def step(k, carry):
        n = vals_v[pl.ds(k * _L, _L)]
        w = (1.0 - _BETA) / (1.0 - jnp.exp(n * _LN_BETA))
        vals_v[pl.ds(k * _L, _L)] = w
        return carry

    lax.fori_loop(0, _EPW // _L, step, 0)

    copies = [
        pltpu.make_async_copy(
            vals_v.at[pl.ds(a * _CPW, _CPW)],
            out_hbm.at[a, pl.ds(c0, _CPW)],
            sem,
        )
        for a in range(_A)
    ]
    for c in copies:
        c.start()
    for c in copies:
        c.wait()


_sc_call = pl.kernel(
    _fair_scaler_body,
    mesh=plsc.VectorSubcoreMesh(core_axis_name="c", subcore_axis_name="s"),
    out_type=jax.ShapeDtypeStruct((_A, _N), jnp.float32),
    scratch_types=[
        pltpu.VMEM((_EPW,), jnp.int32),
        pltpu.VMEM((_EPW,), jnp.float32),
        pltpu.SemaphoreType.DMA,
    ],
)


def kernel(attr, metric_scores):
    return _sc_call(attr.T, metric_scores).T


# speedup vs baseline: 2.3111x; 1.3290x over previous
import math

import jax
import jax.numpy as jnp
from jax import lax
from jax.experimental import pallas as pl
from jax.experimental.pallas import tpu as pltpu
from jax.experimental.pallas import tpu_sc as plsc

_BETA = 0.9
_LN_BETA = math.log(_BETA)

_N, _A = 16384, 26
_NC, _NS = 2, 16
_NW = _NC * _NS
_CPW = _N // _NW
_EPW = _CPW * _A
_L = 16


_NCH = 8
_CHE = _EPW // _NCH
_UNROLL = 4
_CSTEP = _CHE // (_L * _UNROLL)


def _fair_scaler_body(attr_hbm, ms_hbm, out_hbm, idx_v, vals_v,
                      sem_io, sem_g0, sem_g1):
    wid = lax.axis_index("s") * _NC + lax.axis_index("c")
    c0 = wid * _CPW
    copies = [
        pltpu.make_async_copy(
            attr_hbm.at[a, pl.ds(c0, _CPW)],
            idx_v.at[pl.ds(a * _CPW, _CPW)],
            sem_io,
        )
        for a in range(_A)
    ]
    for c in copies:
        c.start()
    for c in copies:
        c.wait()

    sems = (sem_g0, sem_g1)
    gathers = [
        pltpu.make_async_copy(
            ms_hbm.at[idx_v.at[pl.ds(c * _CHE, _CHE)]],
            vals_v.at[pl.ds(c * _CHE, _CHE)],
            sems[c % 2],
        )
        for c in range(_NCH)
    ]
    gathers[0].start()
    gathers[1].start()
    for c in range(_NCH):
        gathers[c].wait()
        if c + 2 < _NCH:
            gathers[c + 2].start()

        def step(k, carry, base=c * _CHE):
            for j in range(_UNROLL):
                o = base + k * (_L * _UNROLL) + j * _L
                n = vals_v[pl.ds(o, _L)]
                w = (1.0 - _BETA) / (1.0 - jnp.exp(n * _LN_BETA))
                vals_v[pl.ds(o, _L)] = w
            return carry

        lax.fori_loop(0, _CSTEP, step, 0)

    copies = [
        pltpu.make_async_copy(
            vals_v.at[pl.ds(a * _CPW, _CPW)],
            out_hbm.at[a, pl.ds(c0, _CPW)],
            sem_io,
        )
        for a in range(_A)
    ]
    for c in copies:
        c.start()
    for c in copies:
        c.wait()


_sc_call = pl.kernel(
    _fair_scaler_body,
    mesh=plsc.VectorSubcoreMesh(core_axis_name="c", subcore_axis_name="s"),
    out_type=jax.ShapeDtypeStruct((_A, _N), jnp.float32),
    scratch_types=[
        pltpu.VMEM((_EPW,), jnp.int32),
        pltpu.VMEM((_EPW,), jnp.float32),
        pltpu.SemaphoreType.DMA,
        pltpu.SemaphoreType.DMA,
        pltpu.SemaphoreType.DMA,
    ],
)


def kernel(attr, metric_scores):
    return _sc_call(attr.T, metric_scores).T
